# baseline (speedup 1.0000x reference)
import functools

import jax
import jax.numpy as jnp
from jax import lax
from jax.experimental import pallas as pl
from jax.experimental.pallas import tpu as pltpu
from jax.experimental.pallas import tpu_sc as plsc

KV = 512
RD = 64
ROW = KV + RD
NSLOTS = 256 * 256
NTOK = 32 * 256
PS = 256
L = 16
NW = 32
SLOTS_PER_W = NSLOTS // NW
COPY_CHUNK = 512

_mesh = plsc.VectorSubcoreMesh(core_axis_name="c", subcore_axis_name="s")


@functools.partial(
    pl.kernel,
    out_type=jax.ShapeDtypeStruct((NSLOTS, ROW), jnp.float32),
    mesh=_mesh,
    compiler_params=pltpu.CompilerParams(needs_layout_passes=False),
    scratch_types=[
        pltpu.VMEM((NTOK,), jnp.int32),
        pltpu.VMEM((NTOK,), jnp.int32),
        pltpu.VMEM((NSLOTS,), jnp.int32),
        pltpu.SemaphoreType.DMA,
        pltpu.SemaphoreType.DMA,
    ],
)
def _sc_scatter(ks, vs, pi, po, kc, out, pi_v, po_v, w_v, sem_copy, sem_hit):
    wid = lax.axis_index("s") * 2 + lax.axis_index("c")
    base_slot = wid * SLOTS_PER_W

    for c in range(SLOTS_PER_W // COPY_CHUNK):
        r0 = base_slot + c * COPY_CHUNK
        pltpu.async_copy(
            kc.at[pl.ds(r0, COPY_CHUNK), :],
            out.at[pl.ds(r0, COPY_CHUNK), :],
            sem_copy,
        )

    pltpu.sync_copy(pi, pi_v)
    pltpu.sync_copy(po, po_v)

    def _init(i, carry):
        w_v[pl.ds(i * L, L)] = jnp.full((L,), -1, jnp.int32)
        return carry

    lax.fori_loop(0, NSLOTS // L, _init, 0)

    def _winner(g, carry):
        flat = pi_v[pl.ds(g * L, L)] * PS + po_v[pl.ds(g * L, L)]
        tok = lax.iota(jnp.int32, L) + g * L
        _, last = plsc.scan_count(flat)
        plsc.store_scatter(w_v, [flat], tok, mask=last)
        return carry

    lax.fori_loop(0, NTOK // L, _winner, 0)

    for c in range(SLOTS_PER_W // COPY_CHUNK):
        pltpu.make_async_copy(
            kc.at[pl.ds(base_slot, COPY_CHUNK), :],
            out.at[pl.ds(base_slot, COPY_CHUNK), :],
            sem_copy,
        ).wait()

    def _hits(g, nhit):
        s0 = base_slot + g * L
        wv = w_v[pl.ds(s0, L)]
        for lane in range(L):
            w = wv[lane]

            @pl.when(w >= 0)
            def _(w=w, lane=lane):
                pltpu.async_copy(
                    ks.at[pl.ds(w, 1), :],
                    out.at[pl.ds(s0 + lane, 1), pl.ds(0, KV)],
                    sem_hit,
                )
                pltpu.async_copy(
                    vs.at[pl.ds(w, 1), :],
                    out.at[pl.ds(s0 + lane, 1), pl.ds(KV, RD)],
                    sem_hit,
                )

        return nhit + jnp.sum(jnp.where(wv >= 0, jnp.int32(1), jnp.int32(0)))

    nhits = lax.fori_loop(0, SLOTS_PER_W // L, _hits, jnp.int32(0))

    def _drain(i, carry):
        pltpu.make_async_copy(
            ks.at[pl.ds(0, 1), :], out.at[pl.ds(0, 1), pl.ds(0, KV)], sem_hit
        ).wait()
        pltpu.make_async_copy(
            vs.at[pl.ds(0, 1), :], out.at[pl.ds(0, 1), pl.ds(KV, RD)], sem_hit
        ).wait()
        return carry

    lax.fori_loop(0, nhits, _drain, 0)


def kernel(key_states, value_states, layer_idx, page_idx, page_offset, k_cache):
    ks = key_states.reshape(NTOK, KV)
    vs = value_states.reshape(NTOK, RD)
    pi = page_idx.astype(jnp.int32)
    po = page_offset.astype(jnp.int32)
    kc = k_cache.reshape(NSLOTS, ROW)
    out = _sc_scatter(ks, vs, pi, po, kc)
    return out.reshape(k_cache.shape)


# speedup vs baseline: 1.1096x; 1.1096x over previous
import functools

import jax
import jax.numpy as jnp
from jax import lax
from jax.experimental import pallas as pl
from jax.experimental.pallas import tpu as pltpu
from jax.experimental.pallas import tpu_sc as plsc

KV = 512
RD = 64
ROW = KV + RD
NSLOTS = 256 * 256
NTOK = 32 * 256
PS = 256
L = 16
NW = 32
SLOTS_PER_W = NSLOTS // NW
COPY_CHUNK = 512

_mesh = plsc.VectorSubcoreMesh(core_axis_name="c", subcore_axis_name="s")


@functools.partial(
    pl.kernel,
    out_type=jax.ShapeDtypeStruct((NSLOTS, ROW), jnp.float32),
    mesh=_mesh,
    compiler_params=pltpu.CompilerParams(needs_layout_passes=False),
    scratch_types=[
        pltpu.VMEM((NTOK,), jnp.int32),
        pltpu.VMEM((NTOK,), jnp.int32),
        pltpu.VMEM((NSLOTS,), jnp.int32),
        pltpu.SemaphoreType.DMA,
        pltpu.SemaphoreType.DMA,
    ],
)
def _sc_scatter(ks, vs, pi, po, kc, out, pi_v, po_v, w_v, sem_copy, sem_hit):
    wid = lax.axis_index("s") * 2 + lax.axis_index("c")
    base_slot = wid * SLOTS_PER_W

    for c in range(SLOTS_PER_W // COPY_CHUNK):
        r0 = base_slot + c * COPY_CHUNK
        pltpu.async_copy(
            kc.at[pl.ds(r0, COPY_CHUNK), :],
            out.at[pl.ds(r0, COPY_CHUNK), :],
            sem_copy,
        )

    for c in range(SLOTS_PER_W // COPY_CHUNK):
        pltpu.make_async_copy(
            kc.at[pl.ds(base_slot, COPY_CHUNK), :],
            out.at[pl.ds(base_slot, COPY_CHUNK), :],
            sem_copy,
        ).wait()
    return

    pltpu.sync_copy(pi, pi_v)
    pltpu.sync_copy(po, po_v)

    def _init(i, carry):
        w_v[pl.ds(i * L, L)] = jnp.full((L,), -1, jnp.int32)
        return carry

    lax.fori_loop(0, NSLOTS // L, _init, 0)

    def _winner(g, carry):
        flat = pi_v[pl.ds(g * L, L)] * PS + po_v[pl.ds(g * L, L)]
        tok = lax.iota(jnp.int32, L) + g * L
        _, last = plsc.scan_count(flat)
        plsc.store_scatter(w_v, [flat], tok, mask=last)
        return carry

    lax.fori_loop(0, NTOK // L, _winner, 0)

    for c in range(SLOTS_PER_W // COPY_CHUNK):
        pltpu.make_async_copy(
            kc.at[pl.ds(base_slot, COPY_CHUNK), :],
            out.at[pl.ds(base_slot, COPY_CHUNK), :],
            sem_copy,
        ).wait()

    def _hits(g, nhit):
        s0 = base_slot + g * L
        wv = w_v[pl.ds(s0, L)]
        for lane in range(L):
            w = wv[lane]

            @pl.when(w >= 0)
            def _(w=w, lane=lane):
                pltpu.async_copy(
                    ks.at[pl.ds(w, 1), :],
                    out.at[pl.ds(s0 + lane, 1), pl.ds(0, KV)],
                    sem_hit,
                )
                pltpu.async_copy(
                    vs.at[pl.ds(w, 1), :],
                    out.at[pl.ds(s0 + lane, 1), pl.ds(KV, RD)],
                    sem_hit,
                )

        return nhit + jnp.sum(jnp.where(wv >= 0, jnp.int32(1), jnp.int32(0)))

    nhits = lax.fori_loop(0, SLOTS_PER_W // L, _hits, jnp.int32(0))

    def _drain(i, carry):
        pltpu.make_async_copy(
            ks.at[pl.ds(0, 1), :], out.at[pl.ds(0, 1), pl.ds(0, KV)], sem_hit
        ).wait()
        pltpu.make_async_copy(
            vs.at[pl.ds(0, 1), :], out.at[pl.ds(0, 1), pl.ds(KV, RD)], sem_hit
        ).wait()
        return carry

    lax.fori_loop(0, nhits, _drain, 0)


def kernel(key_states, value_states, layer_idx, page_idx, page_offset, k_cache):
    ks = key_states.reshape(NTOK, KV)
    vs = value_states.reshape(NTOK, RD)
    pi = page_idx.astype(jnp.int32)
    po = page_offset.astype(jnp.int32)
    kc = k_cache.reshape(NSLOTS, ROW)
    out = _sc_scatter(ks, vs, pi, po, kc)
    return out.reshape(k_cache.shape)


# speedup vs baseline: 5.8346x; 5.2584x over previous
import functools

import jax
import jax.numpy as jnp
from jax import lax
from jax.experimental import pallas as pl
from jax.experimental.pallas import tpu as pltpu
from jax.experimental.pallas import tpu_sc as plsc

KV = 512
RD = 64
ROW = KV + RD
NSLOTS = 256 * 256
NTOK = 32 * 256
PS = 256
L = 16
NW = 32
SLOTS_PER_W = NSLOTS // NW
CCH = 32
NCH = SLOTS_PER_W // CCH

_mesh = plsc.VectorSubcoreMesh(core_axis_name="c", subcore_axis_name="s")


@functools.partial(
    pl.kernel,
    out_type=jax.ShapeDtypeStruct((NSLOTS, ROW), jnp.float32),
    mesh=_mesh,
    compiler_params=pltpu.CompilerParams(needs_layout_passes=False),
    scratch_types=[
        pltpu.VMEM((NTOK,), jnp.int32),
        pltpu.VMEM((NTOK,), jnp.int32),
        pltpu.VMEM((NSLOTS,), jnp.int32),
        pltpu.VMEM((2, CCH, ROW), jnp.float32),
        pltpu.SemaphoreType.DMA,
        pltpu.SemaphoreType.DMA,
        pltpu.SemaphoreType.DMA,
        pltpu.SemaphoreType.DMA,
    ],
)
def _sc_scatter(ks, vs, pi, po, kc, out, pi_v, po_v, w_v, cb, sem_g, sem_s,
                sem_ix, sem_hit):
    wid = lax.axis_index("s") * 2 + lax.axis_index("c")
    base_slot = wid * SLOTS_PER_W

    pltpu.async_copy(pi, pi_v, sem_ix)
    pltpu.async_copy(po, po_v, sem_ix)

    def _init(i, carry):
        w_v[pl.ds(i * L, L)] = jnp.full((L,), -1, jnp.int32)
        return carry

    lax.fori_loop(0, NSLOTS // L, _init, 0)
    pltpu.make_async_copy(pi, pi_v, sem_ix).wait()
    pltpu.make_async_copy(po, po_v, sem_ix).wait()

    def _winner(g, carry):
        flat = pi_v[pl.ds(g * L, L)] * PS + po_v[pl.ds(g * L, L)]
        tok = lax.iota(jnp.int32, L) + g * L
        _, last = plsc.scan_count(flat)
        plsc.store_scatter(w_v, [flat], tok, mask=last)
        return carry

    lax.fori_loop(0, NTOK // L, _winner, 0)

    def _g_copy(c, buf):
        r0 = base_slot + c * CCH
        return pltpu.make_async_copy(kc.at[pl.ds(r0, CCH), :], cb.at[buf], sem_g)

    def _s_copy(c, buf):
        r0 = base_slot + c * CCH
        return pltpu.make_async_copy(cb.at[buf], out.at[pl.ds(r0, CCH), :], sem_s)

    _g_copy(0, 0).start()

    def _copy_body(c2, carry):
        c = c2 * 2
        _g_copy(c, 0).wait()
        _s_copy(c, 0).start()

        @pl.when(c2 > 0)
        def _():
            _s_copy(c - 1, 1).wait()

        _g_copy(c + 1, 1).start()

        _g_copy(c + 1, 1).wait()
        _s_copy(c + 1, 1).start()
        _s_copy(c, 0).wait()

        @pl.when(c2 < NCH // 2 - 1)
        def _():
            _g_copy(c + 2, 0).start()

        return carry

    lax.fori_loop(0, NCH // 2, _copy_body, 0)
    _s_copy(NCH - 1, 1).wait()

    def _hits(g, nhit):
        s0 = base_slot + g * L
        wv = w_v[pl.ds(s0, L)]
        for lane in range(L):
            w = wv[lane]

            @pl.when(w >= 0)
            def _(w=w, lane=lane):
                pltpu.async_copy(
                    ks.at[pl.ds(w, 1), :],
                    out.at[pl.ds(s0 + lane, 1), pl.ds(0, KV)],
                    sem_hit,
                )
                pltpu.async_copy(
                    vs.at[pl.ds(w, 1), :],
                    out.at[pl.ds(s0 + lane, 1), pl.ds(KV, RD)],
                    sem_hit,
                )

        return nhit + jnp.sum(jnp.where(wv >= 0, jnp.int32(1), jnp.int32(0)))

    nhits = lax.fori_loop(0, SLOTS_PER_W // L, _hits, jnp.int32(0))

    def _drain(i, carry):
        pltpu.make_async_copy(
            ks.at[pl.ds(0, 1), :], out.at[pl.ds(0, 1), pl.ds(0, KV)], sem_hit
        ).wait()
        pltpu.make_async_copy(
            vs.at[pl.ds(0, 1), :], out.at[pl.ds(0, 1), pl.ds(KV, RD)], sem_hit
        ).wait()
        return carry

    lax.fori_loop(0, nhits, _drain, 0)


def kernel(key_states, value_states, layer_idx, page_idx, page_offset, k_cache):
    ks = key_states.reshape(NTOK, KV)
    vs = value_states.reshape(NTOK, RD)
    pi = page_idx.astype(jnp.int32)
    po = page_offset.astype(jnp.int32)
    kc = k_cache.reshape(NSLOTS, ROW)
    out = _sc_scatter(ks, vs, pi, po, kc)
    return out.reshape(k_cache.shape)


# speedup vs baseline: 6.0323x; 1.0339x over previous
import functools

import jax
import jax.numpy as jnp
from jax import lax
from jax.experimental import pallas as pl
from jax.experimental.pallas import tpu as pltpu
from jax.experimental.pallas import tpu_sc as plsc

KV = 512
RD = 64
ROW = KV + RD
NSLOTS = 256 * 256
NTOK = 32 * 256
PS = 256
L = 16
NW = 32
SLOTS_PER_W = NSLOTS // NW
CCH = 32
NCH = SLOTS_PER_W // CCH
NB = 4

_mesh = plsc.VectorSubcoreMesh(core_axis_name="c", subcore_axis_name="s")


@functools.partial(
    pl.kernel,
    out_type=jax.ShapeDtypeStruct((NSLOTS, ROW), jnp.float32),
    mesh=_mesh,
    compiler_params=pltpu.CompilerParams(needs_layout_passes=False),
    scratch_types=[
        pltpu.VMEM((NTOK,), jnp.int32),
        pltpu.VMEM((NTOK,), jnp.int32),
        pltpu.VMEM((SLOTS_PER_W,), jnp.int32),
        pltpu.VMEM((NB * CCH, ROW), jnp.float32),
        pltpu.SemaphoreType.DMA,
        pltpu.SemaphoreType.DMA,
        pltpu.SemaphoreType.DMA,
        pltpu.SemaphoreType.DMA,
    ],
)
def _sc_scatter(ks, vs, pi, po, kc, out, pi_v, po_v, w_v, cb, sem_g, sem_s,
                sem_ix, sem_hit):
    wid = lax.axis_index("s") * 2 + lax.axis_index("c")
    base_slot = wid * SLOTS_PER_W

    pltpu.async_copy(pi, pi_v, sem_ix)
    pltpu.async_copy(po, po_v, sem_ix)

    def _init(i, carry):
        w_v[pl.ds(i * L, L)] = jnp.full((L,), -1, jnp.int32)
        return carry

    lax.fori_loop(0, SLOTS_PER_W // L, _init, 0)
    pltpu.make_async_copy(pi, pi_v, sem_ix).wait()
    pltpu.make_async_copy(po, po_v, sem_ix).wait()

    def _winner(g, carry):
        flat = pi_v[pl.ds(g * L, L)] * PS + po_v[pl.ds(g * L, L)]
        tok = lax.iota(jnp.int32, L) + g * L
        _, last = plsc.scan_count(flat)
        rel = flat - base_slot
        m = last & (rel >= 0) & (rel < SLOTS_PER_W)
        rel_safe = jnp.where(m, rel, 0)
        plsc.store_scatter(w_v, [rel_safe], tok, mask=m)
        return carry

    lax.fori_loop(0, NTOK // L, _winner, 0)

    def _g_copy(c, buf):
        r0 = base_slot + c * CCH
        return pltpu.make_async_copy(
            kc.at[pl.ds(r0, CCH), :], cb.at[pl.ds(buf * CCH, CCH), :], sem_g
        )

    def _s_copy(c, buf):
        r0 = base_slot + c * CCH
        return pltpu.make_async_copy(
            cb.at[pl.ds(buf * CCH, CCH), :], out.at[pl.ds(r0, CCH), :], sem_s
        )

    for b in range(NB):
        _g_copy(b, b).start()

    def _copy_body(c4, carry):
        for b in range(NB):
            c = c4 * NB + b
            _g_copy(c, b).wait()
            _s_copy(c, b).start()

            @pl.when((c >= 1) & (c < NCH - NB + 1))
            def _(c=c, b=b):
                _s_copy(c - 1, (b + NB - 1) % NB).wait()
                _g_copy(c + NB - 1, (b + NB - 1) % NB).start()

        return carry

    lax.fori_loop(0, NCH // NB, _copy_body, 0)
    for i in range(NB):
        _s_copy(NCH - NB + i, (NCH - NB + i) % NB).wait()

    def _hits(g, nhit):
        wv = w_v[pl.ds(g * L, L)]
        s0 = base_slot + g * L
        for lane in range(L):
            w = wv[lane]

            @pl.when(w >= 0)
            def _(w=w, lane=lane):
                pltpu.async_copy(
                    ks.at[pl.ds(w, 1), :],
                    out.at[pl.ds(s0 + lane, 1), pl.ds(0, KV)],
                    sem_hit,
                )
                pltpu.async_copy(
                    vs.at[pl.ds(w, 1), :],
                    out.at[pl.ds(s0 + lane, 1), pl.ds(KV, RD)],
                    sem_hit,
                )

        return nhit + jnp.sum(jnp.where(wv >= 0, jnp.int32(1), jnp.int32(0)))

    nhits = lax.fori_loop(0, SLOTS_PER_W // L, _hits, jnp.int32(0))

    def _drain(i, carry):
        pltpu.make_async_copy(
            ks.at[pl.ds(0, 1), :], out.at[pl.ds(0, 1), pl.ds(0, KV)], sem_hit
        ).wait()
        pltpu.make_async_copy(
            vs.at[pl.ds(0, 1), :], out.at[pl.ds(0, 1), pl.ds(KV, RD)], sem_hit
        ).wait()
        return carry

    lax.fori_loop(0, nhits, _drain, 0)


def kernel(key_states, value_states, layer_idx, page_idx, page_offset, k_cache):
    ks = key_states.reshape(NTOK, KV)
    vs = value_states.reshape(NTOK, RD)
    pi = page_idx.astype(jnp.int32)
    po = page_offset.astype(jnp.int32)
    kc = k_cache.reshape(NSLOTS, ROW)
    out = _sc_scatter(ks, vs, pi, po, kc)
    return out.reshape(k_cache.shape)


# speedup vs baseline: 6.7936x; 1.1262x over previous
import functools

import jax
import jax.numpy as jnp
from jax import lax
from jax.experimental import pallas as pl
from jax.experimental.pallas import tpu as pltpu
from jax.experimental.pallas import tpu_sc as plsc

KV = 512
RD = 64
ROW = KV + RD
NSLOTS = 256 * 256
NTOK = 32 * 256
PS = 256
L = 16
NW = 32
SLOTS_PER_W = NSLOTS // NW

_mesh = plsc.VectorSubcoreMesh(core_axis_name="c", subcore_axis_name="s")


@functools.partial(
    pl.kernel,
    out_type=(),
    mesh=_mesh,
    compiler_params=pltpu.CompilerParams(needs_layout_passes=False),
    scratch_types=[
        pltpu.VMEM((NTOK,), jnp.int32),
        pltpu.VMEM((NTOK,), jnp.int32),
        pltpu.VMEM((SLOTS_PER_W,), jnp.int32),
        pltpu.SemaphoreType.DMA,
        pltpu.SemaphoreType.DMA,
    ],
)
def _sc_scatter(ks, vs, pi, po, kc, pi_v, po_v, w_v, sem_ix, sem_hit):
    wid = lax.axis_index("s") * 2 + lax.axis_index("c")
    base_slot = wid * SLOTS_PER_W

    pltpu.async_copy(pi, pi_v, sem_ix)
    pltpu.async_copy(po, po_v, sem_ix)

    def _init(i, carry):
        w_v[pl.ds(i * L, L)] = jnp.full((L,), -1, jnp.int32)
        return carry

    lax.fori_loop(0, SLOTS_PER_W // L, _init, 0)
    pltpu.make_async_copy(pi, pi_v, sem_ix).wait()
    pltpu.make_async_copy(po, po_v, sem_ix).wait()

    def _winner(g, carry):
        flat = pi_v[pl.ds(g * L, L)] * PS + po_v[pl.ds(g * L, L)]
        tok = lax.iota(jnp.int32, L) + g * L
        _, last = plsc.scan_count(flat)
        rel = flat - base_slot
        m = last & (rel >= 0) & (rel < SLOTS_PER_W)
        rel_safe = jnp.where(m, rel, 0)
        plsc.store_scatter(w_v, [rel_safe], tok, mask=m)
        return carry

    lax.fori_loop(0, NTOK // L, _winner, 0)

    def _hits(g, nhit):
        wv = w_v[pl.ds(g * L, L)]
        s0 = base_slot + g * L
        for lane in range(L):
            w = wv[lane]

            @pl.when(w >= 0)
            def _(w=w, lane=lane):
                pltpu.async_copy(
                    ks.at[pl.ds(w, 1), :],
                    kc.at[pl.ds(s0 + lane, 1), pl.ds(0, KV)],
                    sem_hit,
                )
                pltpu.async_copy(
                    vs.at[pl.ds(w, 1), :],
                    kc.at[pl.ds(s0 + lane, 1), pl.ds(KV, RD)],
                    sem_hit,
                )

        return nhit + jnp.sum(jnp.where(wv >= 0, jnp.int32(1), jnp.int32(0)))

    nhits = lax.fori_loop(0, SLOTS_PER_W // L, _hits, jnp.int32(0))

    def _drain(i, carry):
        pltpu.make_async_copy(
            ks.at[pl.ds(0, 1), :], kc.at[pl.ds(0, 1), pl.ds(0, KV)], sem_hit
        ).wait()
        pltpu.make_async_copy(
            vs.at[pl.ds(0, 1), :], kc.at[pl.ds(0, 1), pl.ds(KV, RD)], sem_hit
        ).wait()
        return carry

    lax.fori_loop(0, nhits, _drain, 0)


def kernel(key_states, value_states, layer_idx, page_idx, page_offset, k_cache):
    ks = key_states.reshape(NTOK, KV)
    vs = value_states.reshape(NTOK, RD)
    pi = page_idx.astype(jnp.int32)
    po = page_offset.astype(jnp.int32)
    kc_ref = jax.new_ref(k_cache.reshape(NSLOTS, ROW))
    _sc_scatter(ks, vs, pi, po, kc_ref)
    return kc_ref[...].reshape(k_cache.shape)


# speedup vs baseline: 13.5980x; 2.0016x over previous
import functools

import jax
import jax.numpy as jnp
from jax import lax
from jax.experimental import pallas as pl
from jax.experimental.pallas import tpu as pltpu
from jax.experimental.pallas import tpu_sc as plsc

KV = 512
RD = 64
ROW = KV + RD
NSLOTS = 256 * 256
NTOK = 32 * 256
PS = 256
L = 16
NW = 32
SLOTS_PER_W = NSLOTS // NW
CH = 128
MAXCH = SLOTS_PER_W // CH

_mesh = plsc.VectorSubcoreMesh(core_axis_name="c", subcore_axis_name="s")


@functools.partial(
    pl.kernel,
    out_type=(),
    mesh=_mesh,
    compiler_params=pltpu.CompilerParams(needs_layout_passes=False),
    scratch_types=[
        pltpu.VMEM((NTOK,), jnp.int32),
        pltpu.VMEM((NTOK,), jnp.int32),
        pltpu.VMEM((SLOTS_PER_W,), jnp.int32),
        pltpu.VMEM((SLOTS_PER_W,), jnp.int32),
        pltpu.VMEM((SLOTS_PER_W,), jnp.int32),
        pltpu.VMEM((MAXCH, CH), jnp.int32),
        pltpu.VMEM((CH, KV), jnp.float32),
        pltpu.SemaphoreType.DMA,
        pltpu.SemaphoreType.DMA,
        pltpu.SemaphoreType.DMA,
    ],
)
def _sc_scatter(ksv, pi, po, kc, pi_v, po_v, w_v, src_f, dst_f, d2, stk,
                sem_ix, sem_g, sem_s):
    wid = lax.axis_index("s") * 2 + lax.axis_index("c")
    base_slot = wid * SLOTS_PER_W

    pltpu.async_copy(pi, pi_v, sem_ix)
    pltpu.async_copy(po, po_v, sem_ix)

    def _init(i, carry):
        w_v[pl.ds(i * L, L)] = jnp.full((L,), -1, jnp.int32)
        return carry

    lax.fori_loop(0, SLOTS_PER_W // L, _init, 0)
    pltpu.make_async_copy(pi, pi_v, sem_ix).wait()
    pltpu.make_async_copy(po, po_v, sem_ix).wait()

    def _winner(g, carry):
        flat = pi_v[pl.ds(g * L, L)] * PS + po_v[pl.ds(g * L, L)]
        tok = lax.iota(jnp.int32, L) + g * L
        _, last = plsc.scan_count(flat)
        rel = flat - base_slot
        m = last & (rel >= 0) & (rel < SLOTS_PER_W)
        rel_safe = jnp.where(m, rel, 0)
        plsc.store_scatter(w_v, [rel_safe], tok, mask=m)
        return carry

    lax.fori_loop(0, NTOK // L, _winner, 0)

    def _compact(g, off):
        wv = w_v[pl.ds(g * L, L)]
        m = wv >= 0
        slots = lax.iota(jnp.int32, L) + (base_slot + g * L)
        plsc.store_compressed(src_f.at[pl.ds(off, L)], wv, mask=m)
        plsc.store_compressed(dst_f.at[pl.ds(off, L)], slots, mask=m)
        return off + jnp.sum(jnp.where(m, jnp.int32(1), jnp.int32(0)))

    nhit = lax.fori_loop(0, SLOTS_PER_W // L, _compact, jnp.int32(0))
    nch = (nhit + (CH - 1)) // CH

    @pl.when(nhit > 0)
    def _():
        d0 = dst_f[pl.ds(0, L)]
        s0 = src_f[pl.ds(0, L)]
        pad_d = jnp.full((L,), d0[0], jnp.int32)
        pad_s = jnp.full((L,), s0[0], jnp.int32)

        def _pad(g, carry):
            idx = lax.iota(jnp.int32, L) + g * L
            m = idx >= nhit
            dv = dst_f[pl.ds(g * L, L)]
            sv = src_f[pl.ds(g * L, L)]
            dst_f[pl.ds(g * L, L)] = jnp.where(m, pad_d, dv)
            src_f[pl.ds(g * L, L)] = jnp.where(m, pad_s, sv)
            return carry

        lax.fori_loop(nhit // L, (nch * CH) // L, _pad, 0)

        def _t(r, carry):
            for cg in range(CH // L):
                d2[r, pl.ds(cg * L, L)] = dst_f[pl.ds(r * CH + cg * L, L)]
            return carry

        lax.fori_loop(0, nch, _t, 0)

        def _vals(g, carry):
            dv = dst_f[pl.ds(g * L, L)]
            sv = src_f[pl.ds(g * L, L)]
            base = g * L
            for lane in range(L):
                @pl.when(base + lane < nhit)
                def _(d=dv[lane], s=sv[lane]):
                    pltpu.async_copy(
                        ksv.at[pl.ds(s, 1), pl.ds(KV, RD)],
                        kc.at[pl.ds(d, 1), pl.ds(KV, RD)],
                        sem_s,
                    )
            return carry

        nvg = (nhit + (L - 1)) // L
        lax.fori_loop(0, nvg, _vals, 0)

        def _chunk(j, carry):
            src_sl = src_f.at[pl.ds(j * CH, CH)]
            pltpu.async_copy(ksv.at[src_sl, pl.ds(0, KV)], stk, sem_g)
            pltpu.make_async_copy(ksv.at[src_sl, pl.ds(0, KV)], stk, sem_g).wait()
            pltpu.async_copy(stk, kc.at[d2.at[j], pl.ds(0, KV)], sem_g)
            pltpu.make_async_copy(stk, kc.at[d2.at[j], pl.ds(0, KV)], sem_g).wait()
            return carry

        lax.fori_loop(0, nch, _chunk, 0)

        def _drain(i, carry):
            pltpu.make_async_copy(
                ksv.at[pl.ds(0, 1), pl.ds(KV, RD)],
                kc.at[pl.ds(0, 1), pl.ds(KV, RD)],
                sem_s,
            ).wait()
            return carry

        lax.fori_loop(0, nhit, _drain, 0)


def kernel(key_states, value_states, layer_idx, page_idx, page_offset, k_cache):
    ksv = jnp.concatenate(
        [key_states.reshape(NTOK, KV), value_states.reshape(NTOK, RD)], axis=1
    )
    pi = page_idx.astype(jnp.int32)
    po = page_offset.astype(jnp.int32)
    kc_ref = jax.new_ref(k_cache.reshape(NSLOTS, ROW))
    _sc_scatter(ksv, pi, po, kc_ref)
    return kc_ref[...].reshape(k_cache.shape)


# speedup vs baseline: 13.9502x; 1.0259x over previous
import functools

import jax
import jax.numpy as jnp
from jax import lax
from jax.experimental import pallas as pl
from jax.experimental.pallas import tpu as pltpu
from jax.experimental.pallas import tpu_sc as plsc

KV = 512
RD = 64
ROW = KV + RD
NSLOTS = 256 * 256
NTOK = 32 * 256
PS = 256
L = 16
NW = 32
SLOTS_PER_W = NSLOTS // NW
CH = 64
MAXCH = SLOTS_PER_W // CH

_mesh = plsc.VectorSubcoreMesh(core_axis_name="c", subcore_axis_name="s")


@functools.partial(
    pl.kernel,
    out_type=(),
    mesh=_mesh,
    compiler_params=pltpu.CompilerParams(needs_layout_passes=False),
    scratch_types=[
        pltpu.VMEM((NTOK,), jnp.int32),
        pltpu.VMEM((NTOK,), jnp.int32),
        pltpu.VMEM((SLOTS_PER_W,), jnp.int32),
        pltpu.VMEM((SLOTS_PER_W,), jnp.int32),
        pltpu.VMEM((SLOTS_PER_W,), jnp.int32),
        pltpu.VMEM((MAXCH, CH), jnp.int32),
        pltpu.VMEM((2 * CH, KV), jnp.float32),
        pltpu.SemaphoreType.DMA,
        pltpu.SemaphoreType.DMA,
        pltpu.SemaphoreType.DMA,
        pltpu.SemaphoreType.DMA,
    ],
)
def _sc_scatter(ksv, pi, po, kc, pi_v, po_v, w_v, src_f, dst_f, d2, stk,
                sem_ix, sem_g, sem_k, sem_v):
    wid = lax.axis_index("s") * 2 + lax.axis_index("c")
    base_slot = wid * SLOTS_PER_W

    pltpu.async_copy(pi, pi_v, sem_ix)
    pltpu.async_copy(po, po_v, sem_ix)

    def _init(i, carry):
        w_v[pl.ds(i * L, L)] = jnp.full((L,), -1, jnp.int32)
        return carry

    lax.fori_loop(0, SLOTS_PER_W // L, _init, 0)
    pltpu.make_async_copy(pi, pi_v, sem_ix).wait()
    pltpu.make_async_copy(po, po_v, sem_ix).wait()

    def _winner(g, carry):
        flat = pi_v[pl.ds(g * L, L)] * PS + po_v[pl.ds(g * L, L)]
        tok = lax.iota(jnp.int32, L) + g * L
        _, last = plsc.scan_count(flat)
        rel = flat - base_slot
        m = last & (rel >= 0) & (rel < SLOTS_PER_W)
        rel_safe = jnp.where(m, rel, 0)
        plsc.store_scatter(w_v, [rel_safe], tok, mask=m)
        return carry

    lax.fori_loop(0, NTOK // L, _winner, 0)

    def _compact(g, off):
        wv = w_v[pl.ds(g * L, L)]
        m = wv >= 0
        slots = lax.iota(jnp.int32, L) + (base_slot + g * L)
        plsc.store_compressed(src_f.at[pl.ds(off, L)], wv, mask=m)
        plsc.store_compressed(dst_f.at[pl.ds(off, L)], slots, mask=m)
        return off + jnp.sum(jnp.where(m, jnp.int32(1), jnp.int32(0)))

    nhit = lax.fori_loop(0, SLOTS_PER_W // L, _compact, jnp.int32(0))
    nch = (nhit + (CH - 1)) // CH

    @pl.when(nhit > 0)
    def _():
        d0 = dst_f[pl.ds(0, L)]
        s0 = src_f[pl.ds(0, L)]
        pad_d = jnp.full((L,), d0[0], jnp.int32)
        pad_s = jnp.full((L,), s0[0], jnp.int32)

        def _pad(g, carry):
            idx = lax.iota(jnp.int32, L) + g * L
            m = idx >= nhit
            dv = dst_f[pl.ds(g * L, L)]
            sv = src_f[pl.ds(g * L, L)]
            dst_f[pl.ds(g * L, L)] = jnp.where(m, pad_d, dv)
            src_f[pl.ds(g * L, L)] = jnp.where(m, pad_s, sv)
            return carry

        lax.fori_loop(nhit // L, (nch * CH) // L, _pad, 0)

        def _t(r, carry):
            for cg in range(CH // L):
                d2[r, pl.ds(cg * L, L)] = dst_f[pl.ds(r * CH + cg * L, L)]
            return carry

        lax.fori_loop(0, nch, _t, 0)

        def _vals(g, carry):
            dv = dst_f[pl.ds(g * L, L)]
            sv = src_f[pl.ds(g * L, L)]
            base = g * L
            for lane in range(L):
                @pl.when(base + lane < nhit)
                def _(d=dv[lane], s=sv[lane]):
                    pltpu.async_copy(
                        ksv.at[pl.ds(s, 1), pl.ds(KV, RD)],
                        kc.at[pl.ds(d, 1), pl.ds(KV, RD)],
                        sem_v,
                    )
            return carry

        nvg = (nhit + (L - 1)) // L
        lax.fori_loop(0, nvg, _vals, 0)

        def _g(j, b):
            return pltpu.make_async_copy(
                ksv.at[src_f.at[pl.ds(j * CH, CH)], pl.ds(0, KV)],
                stk.at[pl.ds(b * CH, CH), :],
                sem_g,
            )

        def _s(j, b):
            return pltpu.make_async_copy(
                stk.at[pl.ds(b * CH, CH), :],
                kc.at[d2.at[j], pl.ds(0, KV)],
                sem_k,
            )

        _g(0, 0).start()

        def _chunk(j, carry):
            b = j % 2
            _g(j, b).wait()

            @pl.when(j >= 1)
            def _():
                _s(j - 1, 1 - b).wait()

            @pl.when(j + 1 < nch)
            def _():
                _g(j + 1, 1 - b).start()

            _s(j, b).start()
            return carry

        lax.fori_loop(0, nch, _chunk, 0)
        _s(nch - 1, (nch - 1) % 2).wait()

        def _drain(i, carry):
            pltpu.make_async_copy(
                ksv.at[pl.ds(0, 1), pl.ds(KV, RD)],
                kc.at[pl.ds(0, 1), pl.ds(KV, RD)],
                sem_v,
            ).wait()
            return carry

        lax.fori_loop(0, nhit, _drain, 0)


def kernel(key_states, value_states, layer_idx, page_idx, page_offset, k_cache):
    ksv = jnp.concatenate(
        [key_states.reshape(NTOK, KV), value_states.reshape(NTOK, RD)], axis=1
    )
    pi = page_idx.astype(jnp.int32)
    po = page_offset.astype(jnp.int32)
    kc_ref = jax.new_ref(k_cache.reshape(NSLOTS, ROW))
    _sc_scatter(ksv, pi, po, kc_ref)
    return kc_ref[...].reshape(k_cache.shape)
